# baseline (speedup 1.0000x reference)
import functools

import jax
import jax.numpy as jnp
from jax import lax
from jax.experimental import pallas as pl
from jax.experimental.pallas import tpu as pltpu
from jax.experimental.pallas import tpu_sc as plsc

VOCAB = 1000000
D = 64
B = 16384
L = 50
NUM_CLASS = 5

NC = 2
NS = 16
NW = NC * NS
BAGS_PER_W = B // NW
BAGS_PER_CHUNK = 2
ROWS_PER_CHUNK = BAGS_PER_CHUNK * L
NCHUNK = BAGS_PER_W // BAGS_PER_CHUNK
NBUF = 2
NPARW = 4


def _sc_body(idx_hbm, par_hbm, table_hbm, out_hbm, idx_v, par_vm, gbuf,
             outb, sem0, sem1):
    wid = lax.axis_index("s") * NC + lax.axis_index("c")
    row0 = wid * NCHUNK
    pltpu.sync_copy(idx_hbm.at[pl.ds(row0, NCHUNK)], idx_v)
    pltpu.sync_copy(par_hbm.at[pl.ds(row0 * NPARW, NCHUNK * NPARW)],
                    par_vm.at[pl.ds(0, NCHUNK * NPARW)])

    sems = (sem0, sem1)

    def start(j, b):
        pltpu.async_copy(table_hbm.at[idx_v.at[j]], gbuf.at[b], sems[b])

    def wait(b):
        pltpu.make_async_copy(table_hbm.at[idx_v.at[0]], gbuf.at[b], sems[b]).wait()

    for b in range(NBUF):
        start(b, b)

    zeros = jnp.zeros((16,), jnp.float32)

    @pl.loop(0, NCHUNK, step=NBUF)
    def _chunks(g):
        for b in range(NBUF):
            j = g + b
            wait(b)

            for bic in range(BAGS_PER_CHUNK):
                r0 = bic * L

                def body(r, accs):
                    row = r0 + r
                    wv = par_vm[pl.ds(j * NPARW + lax.shift_right_logical(row, 5), 16)]
                    bit = lax.bitwise_and(
                        lax.shift_right_logical(wv[0], lax.bitwise_and(row, 31)), 1)
                    off = bit * 64
                    return tuple(
                        accs[d] + gbuf[b, row, pl.ds(off + d * 16, 16)]
                        for d in range(4)
                    )

                accs = lax.fori_loop(0, L, body, (zeros,) * 4, unroll=2)
                for d in range(4):
                    outb[j, pl.ds(bic * 64 + d * 16, 16)] = accs[d]

            @pl.when(j + NBUF < NCHUNK)
            def _():
                start(j + NBUF, b)

    pltpu.sync_copy(outb, out_hbm.at[pl.ds(row0, NCHUNK)])


@functools.partial(
    pl.kernel,
    out_type=jax.ShapeDtypeStruct((B // BAGS_PER_CHUNK, 2 * D), jnp.float32),
    mesh=plsc.VectorSubcoreMesh(core_axis_name="c", subcore_axis_name="s"),
    scratch_types=[
        pltpu.VMEM((NCHUNK, ROWS_PER_CHUNK), jnp.int32),
        pltpu.VMEM((NCHUNK * NPARW + 16,), jnp.int32),
        pltpu.VMEM((NBUF, ROWS_PER_CHUNK, 2 * D), jnp.float32),
        pltpu.VMEM((NCHUNK, 2 * D), jnp.float32),
        pltpu.SemaphoreType.DMA,
        pltpu.SemaphoreType.DMA,
    ],
    compiler_params=pltpu.CompilerParams(use_tc_tiling_on_sc=True),
)
def _sc_bag_sums(idx_hbm, par_hbm, table_hbm, out_hbm, idx_v, par_vm, gbuf,
                 outb, sem0, sem1):
    _sc_body(idx_hbm, par_hbm, table_hbm, out_hbm, idx_v, par_vm, gbuf,
             outb, sem0, sem1)


def _tc_linear_body(x_ref, w_ref, b_ref, o_ref):
    o_ref[...] = (
        jnp.dot(x_ref[...], w_ref[...], preferred_element_type=jnp.float32)
        + b_ref[...]
    )


def _tc_linear(sums, w_pad, b_pad):
    blk = 2048
    return pl.pallas_call(
        _tc_linear_body,
        grid=(B // blk,),
        in_specs=[
            pl.BlockSpec((blk, D), lambda i: (i, 0)),
            pl.BlockSpec((D, 128), lambda i: (0, 0)),
            pl.BlockSpec((1, 128), lambda i: (0, 0)),
        ],
        out_specs=pl.BlockSpec((blk, 128), lambda i: (i, 0)),
        out_shape=jax.ShapeDtypeStruct((B, 128), jnp.float32),
    )(sums, w_pad, b_pad)


def kernel(text, emb_table, fc_w, fc_b):
    t32 = text.astype(jnp.int32)
    idx2d = lax.shift_right_logical(t32, 1).reshape(NW * NCHUNK, ROWS_PER_CHUNK)
    par = lax.bitwise_and(t32, 1).reshape(NW * NCHUNK, ROWS_PER_CHUNK)
    par = jnp.pad(par, ((0, 0), (0, 32 * NPARW - ROWS_PER_CHUNK)))
    par = par.reshape(NW * NCHUNK, NPARW, 32)
    shifts = jnp.arange(32, dtype=jnp.int32).reshape(1, 1, 32)
    par1d = jnp.sum(par << shifts, axis=-1).astype(jnp.int32).reshape(-1)

    table128 = emb_table.reshape(VOCAB // 2, 2 * D)
    sums = _sc_bag_sums(idx2d, par1d, table128).reshape(B, D)

    w_pad = jnp.zeros((D, 128), jnp.float32).at[:, :NUM_CLASS].set(fc_w.T / L)
    b_pad = jnp.zeros((1, 128), jnp.float32).at[0, :NUM_CLASS].set(fc_b)
    out = _tc_linear(sums, w_pad, b_pad)
    return out[:, :NUM_CLASS]


# speedup vs baseline: 1.3269x; 1.3269x over previous
import functools

import jax
import jax.numpy as jnp
from jax import lax
from jax.experimental import pallas as pl
from jax.experimental.pallas import tpu as pltpu
from jax.experimental.pallas import tpu_sc as plsc

VOCAB = 1000000
D = 64
B = 16384
L = 50
NUM_CLASS = 5

NC = 2
NS = 16
NW = NC * NS
BAGS_PER_W = B // NW
BAGS_PER_CHUNK = 2
ROWS_PER_CHUNK = BAGS_PER_CHUNK * L
NCHUNK = BAGS_PER_W // BAGS_PER_CHUNK
NBUF = 2

VBLK = 2048
N_VBLK = -(-VOCAB // VBLK)
PACK_ROWS = N_VBLK * (VBLK // 8)


def _tc_proj_body(x_ref, w_ref, b_ref, o_ref):
    y = lax.dot_general(x_ref[...], w_ref[...], (((0,), (0,)), ((), ())),
                        preferred_element_type=jnp.float32)
    y = y + b_ref[...]
    pieces = []
    for k in range(8):
        yk = y[256 * k:256 * (k + 1), 0:8]
        pieces.append(yk)
        pieces.append(yk)
    o_ref[...] = jnp.concatenate(pieces, axis=1)


def _tc_proj(table_t, w_pad, b_pad):
    return pl.pallas_call(
        _tc_proj_body,
        grid=(N_VBLK,),
        in_specs=[
            pl.BlockSpec((D, VBLK), lambda i: (0, i)),
            pl.BlockSpec((D, 128), lambda i: (0, 0)),
            pl.BlockSpec((1, 128), lambda i: (0, 0)),
        ],
        out_specs=pl.BlockSpec((VBLK // 8, 128), lambda i: (i, 0)),
        out_shape=jax.ShapeDtypeStruct((PACK_ROWS, 128), jnp.float32),
    )(table_t, w_pad, b_pad)


def _sc_body(idx_hbm, proj_hbm, out_hbm, idx_v, gbuf, outb, sem0, sem1):
    wid = lax.axis_index("s") * NC + lax.axis_index("c")
    row0 = wid * NCHUNK
    pltpu.sync_copy(idx_hbm.at[pl.ds(row0, NCHUNK)], idx_v)

    sems = (sem0, sem1)

    def start(j, b):
        pltpu.async_copy(proj_hbm.at[idx_v.at[j]], gbuf.at[b], sems[b])

    def wait(b):
        pltpu.make_async_copy(proj_hbm.at[idx_v.at[0]], gbuf.at[b], sems[b]).wait()

    for b in range(NBUF):
        start(b, b)

    zeros = jnp.zeros((16,), jnp.float32)
    iota16 = lax.iota(jnp.int32, 16)
    lo_mask = iota16 < 8
    perm_idx = lax.rem(iota16, 8)

    @pl.loop(0, NCHUNK, step=NBUF)
    def _chunks(g):
        for b in range(NBUF):
            j = g + b
            wait(b)

            accs = []
            for bic in range(BAGS_PER_CHUNK):
                r0 = bic * L

                def body(r, acc):
                    return acc + gbuf[b, r0 + r, pl.ds(0, 16)]

                accs.append(lax.fori_loop(0, L, body, zeros, unroll=4))

            hi = lax.gather(
                accs[1], perm_idx[:, None],
                lax.GatherDimensionNumbers(offset_dims=(),
                                           collapsed_slice_dims=(0,),
                                           start_index_map=(0,)),
                slice_sizes=(1,),
                mode=lax.GatherScatterMode.PROMISE_IN_BOUNDS)
            combined = jnp.where(lo_mask, accs[0], hi)
            outb[lax.shift_right_logical(j, 3),
                 pl.ds(lax.bitwise_and(j, 7) * 16, 16)] = combined

            @pl.when(j + NBUF < NCHUNK)
            def _():
                start(j + NBUF, b)

    pltpu.sync_copy(outb, out_hbm.at[pl.ds(wid * (BAGS_PER_W // 16),
                                           BAGS_PER_W // 16)])


@functools.partial(
    pl.kernel,
    out_type=jax.ShapeDtypeStruct((B // 16, 128), jnp.float32),
    mesh=plsc.VectorSubcoreMesh(core_axis_name="c", subcore_axis_name="s",
                                num_cores=NC, num_subcores=NS),
    scratch_types=[
        pltpu.VMEM((NCHUNK, ROWS_PER_CHUNK), jnp.int32),
        pltpu.VMEM((NBUF, ROWS_PER_CHUNK, 16), jnp.float32),
        pltpu.VMEM((BAGS_PER_W // 16, 128), jnp.float32),
        pltpu.SemaphoreType.DMA,
        pltpu.SemaphoreType.DMA,
    ],
    compiler_params=pltpu.CompilerParams(use_tc_tiling_on_sc=False),
)
def _sc_bag_sums(idx_hbm, proj_hbm, out_hbm, idx_v, gbuf, outb, sem0, sem1):
    _sc_body(idx_hbm, proj_hbm, out_hbm, idx_v, gbuf, outb, sem0, sem1)


def kernel(text, emb_table, fc_w, fc_b):
    t32 = text.astype(jnp.int32)
    hidx = (
        lax.shift_right_logical(t32, 11) * 2048
        + lax.bitwise_and(t32, 255) * 8
        + lax.bitwise_and(lax.shift_right_logical(t32, 8), 7)
    )
    hidx2d = hidx.reshape(NW * NCHUNK, ROWS_PER_CHUNK)

    w_pad = jnp.zeros((D, 128), jnp.float32).at[:, :NUM_CLASS].set(fc_w.T / L)
    b_pad = jnp.zeros((1, 128), jnp.float32).at[0, :NUM_CLASS].set(fc_b / L)

    proj = _tc_proj(emb_table.T, w_pad, b_pad)
    proj16 = proj.reshape(PACK_ROWS * 8, 16)

    packed = _sc_bag_sums(hidx2d, proj16)
    return packed.reshape(B, 8)[:, :NUM_CLASS]


# speedup vs baseline: 2.6966x; 2.0323x over previous
import functools

import jax
import jax.numpy as jnp
from jax import lax
from jax.experimental import pallas as pl
from jax.experimental.pallas import tpu as pltpu
from jax.experimental.pallas import tpu_sc as plsc

VOCAB = 1000000
D = 64
B = 16384
L = 50
NUM_CLASS = 5

NC = 2
NS = 16
NW = NC * NS
BAGS_PER_W = B // NW
BAGS_PER_CHUNK = 2
ROWS_PER_CHUNK = BAGS_PER_CHUNK * L
NCHUNK = BAGS_PER_W // BAGS_PER_CHUNK
NBUF = 2

VBLK = 16384
N_VBLK = -(-VOCAB // VBLK)
PIECE = VBLK // 8
PACK_ROWS = N_VBLK * (VBLK // 8)


def _tc_proj_body(x_ref, w_ref, b_ref, o_ref):
    y = lax.dot_general(x_ref[...], w_ref[...], (((0,), (0,)), ((), ())),
                        preferred_element_type=jnp.float32)
    y = y + b_ref[...]
    lane = lax.broadcasted_iota(jnp.int32, (PIECE, 128), 1)
    slot = lax.shift_right_logical(lane, 4)
    acc = jnp.zeros((PIECE, 128), jnp.float32)
    for k in range(8):
        acc = acc + jnp.where(slot == k, y[PIECE * k:PIECE * (k + 1), :], 0.0)
    o_ref[...] = acc


def _tc_proj(table_t, w_pad, b_pad):
    return pl.pallas_call(
        _tc_proj_body,
        grid=(N_VBLK,),
        in_specs=[
            pl.BlockSpec((D, VBLK), lambda i: (0, i)),
            pl.BlockSpec((D, 128), lambda i: (0, 0)),
            pl.BlockSpec((1, 128), lambda i: (0, 0)),
        ],
        out_specs=pl.BlockSpec((VBLK // 8, 128), lambda i: (i, 0)),
        out_shape=jax.ShapeDtypeStruct((PACK_ROWS, 128), jnp.float32),
        compiler_params=pltpu.CompilerParams(
            fuse_transposed_lhs_in_matmul=True),
    )(table_t, w_pad, b_pad)


def _sc_body(idx_hbm, proj_hbm, out_hbm, idx_v, gbuf, outb, sem0, sem1):
    wid = lax.axis_index("s") * NC + lax.axis_index("c")
    row0 = wid * NCHUNK
    pltpu.sync_copy(idx_hbm.at[pl.ds(row0, NCHUNK)], idx_v)

    sems = (sem0, sem1)

    def start(j, b):
        pltpu.async_copy(proj_hbm.at[idx_v.at[j]], gbuf.at[b], sems[b])

    def wait(b):
        pltpu.make_async_copy(proj_hbm.at[idx_v.at[0]], gbuf.at[b], sems[b]).wait()

    for b in range(NBUF):
        start(b, b)

    zeros = jnp.zeros((16,), jnp.float32)
    iota16 = lax.iota(jnp.int32, 16)
    lo_mask = iota16 < 8
    perm_idx = lax.rem(iota16, 8)

    @pl.loop(0, NCHUNK, step=NBUF)
    def _chunks(g):
        for b in range(NBUF):
            j = g + b
            wait(b)

            accs = []
            for bic in range(BAGS_PER_CHUNK):
                r0 = bic * L

                def body(r, acc):
                    return acc + gbuf[b, r0 + r, pl.ds(0, 16)]

                accs.append(lax.fori_loop(0, L, body, zeros, unroll=4))

            hi = lax.gather(
                accs[1], perm_idx[:, None],
                lax.GatherDimensionNumbers(offset_dims=(),
                                           collapsed_slice_dims=(0,),
                                           start_index_map=(0,)),
                slice_sizes=(1,),
                mode=lax.GatherScatterMode.PROMISE_IN_BOUNDS)
            combined = jnp.where(lo_mask, accs[0], hi)
            outb[lax.shift_right_logical(j, 3),
                 pl.ds(lax.bitwise_and(j, 7) * 16, 16)] = combined

            @pl.when(j + NBUF < NCHUNK)
            def _():
                start(j + NBUF, b)

    pltpu.sync_copy(outb, out_hbm.at[pl.ds(wid * (BAGS_PER_W // 16),
                                           BAGS_PER_W // 16)])


@functools.partial(
    pl.kernel,
    out_type=jax.ShapeDtypeStruct((B // 16, 128), jnp.float32),
    mesh=plsc.VectorSubcoreMesh(core_axis_name="c", subcore_axis_name="s",
                                num_cores=NC, num_subcores=NS),
    scratch_types=[
        pltpu.VMEM((NCHUNK, ROWS_PER_CHUNK), jnp.int32),
        pltpu.VMEM((NBUF, ROWS_PER_CHUNK, 16), jnp.float32),
        pltpu.VMEM((BAGS_PER_W // 16, 128), jnp.float32),
        pltpu.SemaphoreType.DMA,
        pltpu.SemaphoreType.DMA,
    ],
    compiler_params=pltpu.CompilerParams(use_tc_tiling_on_sc=False),
)
def _sc_bag_sums(idx_hbm, proj_hbm, out_hbm, idx_v, gbuf, outb, sem0, sem1):
    _sc_body(idx_hbm, proj_hbm, out_hbm, idx_v, gbuf, outb, sem0, sem1)


def kernel(text, emb_table, fc_w, fc_b):
    t32 = text.astype(jnp.int32)
    hidx = (
        lax.bitwise_and(t32, -VBLK)
        + lax.bitwise_and(t32, PIECE - 1) * 8
        + lax.shift_right_logical(lax.bitwise_and(t32, VBLK - 1), 11)
    )
    hidx2d = hidx.reshape(NW * NCHUNK, ROWS_PER_CHUNK)

    w16 = jnp.zeros((D, 16), jnp.float32)
    w16 = w16.at[:, :NUM_CLASS].set(fc_w.T / L)
    w16 = w16.at[:, 8:8 + NUM_CLASS].set(fc_w.T / L)
    w_pad = jnp.tile(w16, (1, 8))
    b16 = jnp.zeros((1, 16), jnp.float32)
    b16 = b16.at[0, :NUM_CLASS].set(fc_b / L)
    b16 = b16.at[0, 8:8 + NUM_CLASS].set(fc_b / L)
    b_pad = jnp.tile(b16, (1, 8))

    proj = _tc_proj(emb_table.T, w_pad, b_pad)
    proj16 = proj.reshape(PACK_ROWS * 8, 16)

    packed = _sc_bag_sums(hidx2d, proj16)
    return packed.reshape(B, 8)[:, :NUM_CLASS]


# speedup vs baseline: 3.1583x; 1.1712x over previous
import functools

import jax
import jax.numpy as jnp
from jax import lax
from jax.experimental import pallas as pl
from jax.experimental.pallas import tpu as pltpu
from jax.experimental.pallas import tpu_sc as plsc

VOCAB = 1000000
D = 64
B = 16384
L = 50
NUM_CLASS = 5

NC = 2
NS = 16
NW = NC * NS
BAGS_PER_W = B // NW
BAGS_PER_CHUNK = 2
ROWS_PER_CHUNK = BAGS_PER_CHUNK * L
NCHUNK = BAGS_PER_W // BAGS_PER_CHUNK
NBUF = 4

VBLK = 16384
N_VBLK = -(-VOCAB // VBLK)
PIECE = VBLK // 8
PACK_ROWS = N_VBLK * (VBLK // 8)


def _tc_proj_body(x_ref, w_ref, b_ref, o_ref):
    y = lax.dot_general(x_ref[...], w_ref[...], (((0,), (0,)), ((), ())),
                        preferred_element_type=jnp.float32)
    y = y + b_ref[...]
    lane = lax.broadcasted_iota(jnp.int32, (PIECE, 128), 1)
    slot = lax.shift_right_logical(lane, 4)
    acc = jnp.zeros((PIECE, 128), jnp.float32)
    for k in range(8):
        acc = acc + jnp.where(slot == k, y[PIECE * k:PIECE * (k + 1), :], 0.0)
    o_ref[...] = acc


def _tc_proj(table_t, w_pad, b_pad):
    return pl.pallas_call(
        _tc_proj_body,
        grid=(N_VBLK,),
        in_specs=[
            pl.BlockSpec((D, VBLK), lambda i: (0, i)),
            pl.BlockSpec((D, 128), lambda i: (0, 0)),
            pl.BlockSpec((1, 128), lambda i: (0, 0)),
        ],
        out_specs=pl.BlockSpec((VBLK // 8, 128), lambda i: (i, 0)),
        out_shape=jax.ShapeDtypeStruct((PACK_ROWS, 128), jnp.float32),
        compiler_params=pltpu.CompilerParams(
            fuse_transposed_lhs_in_matmul=True),
    )(table_t, w_pad, b_pad)


def _sc_body(idx_hbm, proj_hbm, out_hbm, idx_v, gbuf, outb, sem0, sem1,
             sem2, sem3):
    wid = lax.axis_index("s") * NC + lax.axis_index("c")
    row0 = wid * NCHUNK
    pltpu.sync_copy(idx_hbm.at[pl.ds(row0, NCHUNK)], idx_v)

    sems = (sem0, sem1, sem2, sem3)

    def start(j, b):
        pltpu.async_copy(proj_hbm.at[idx_v.at[j]], gbuf.at[b], sems[b])

    def wait(b):
        pltpu.make_async_copy(proj_hbm.at[idx_v.at[0]], gbuf.at[b], sems[b]).wait()

    for b in range(NBUF):
        start(b, b)

    zeros = jnp.zeros((16,), jnp.float32)
    iota16 = lax.iota(jnp.int32, 16)
    lo_mask = iota16 < 8
    perm_idx = lax.rem(iota16, 8)

    @pl.loop(0, NCHUNK, step=NBUF)
    def _chunks(g):
        for b in range(NBUF):
            j = g + b
            wait(b)

            accs = []
            for bic in range(BAGS_PER_CHUNK):
                r0 = bic * L

                def body(r, acc):
                    return acc + gbuf[b, r0 + r, pl.ds(0, 16)]

                accs.append(lax.fori_loop(0, L, body, zeros, unroll=4))

            hi = lax.gather(
                accs[1], perm_idx[:, None],
                lax.GatherDimensionNumbers(offset_dims=(),
                                           collapsed_slice_dims=(0,),
                                           start_index_map=(0,)),
                slice_sizes=(1,),
                mode=lax.GatherScatterMode.PROMISE_IN_BOUNDS)
            combined = jnp.where(lo_mask, accs[0], hi)
            outb[lax.shift_right_logical(j, 3),
                 pl.ds(lax.bitwise_and(j, 7) * 16, 16)] = combined

            @pl.when(j + NBUF < NCHUNK)
            def _():
                start(j + NBUF, b)

    pltpu.sync_copy(outb, out_hbm.at[pl.ds(wid * (BAGS_PER_W // 16),
                                           BAGS_PER_W // 16)])


@functools.partial(
    pl.kernel,
    out_type=jax.ShapeDtypeStruct((B // 16, 128), jnp.float32),
    mesh=plsc.VectorSubcoreMesh(core_axis_name="c", subcore_axis_name="s",
                                num_cores=NC, num_subcores=NS),
    scratch_types=[
        pltpu.VMEM((NCHUNK, ROWS_PER_CHUNK), jnp.int32),
        pltpu.VMEM((NBUF, ROWS_PER_CHUNK, 16), jnp.float32),
        pltpu.VMEM((BAGS_PER_W // 16, 128), jnp.float32),
        pltpu.SemaphoreType.DMA,
        pltpu.SemaphoreType.DMA,
        pltpu.SemaphoreType.DMA,
        pltpu.SemaphoreType.DMA,
    ],
    compiler_params=pltpu.CompilerParams(use_tc_tiling_on_sc=False),
)
def _sc_bag_sums(idx_hbm, proj_hbm, out_hbm, idx_v, gbuf, outb, sem0, sem1,
                 sem2, sem3):
    _sc_body(idx_hbm, proj_hbm, out_hbm, idx_v, gbuf, outb, sem0, sem1,
             sem2, sem3)


def kernel(text, emb_table, fc_w, fc_b):
    t32 = text.astype(jnp.int32)
    hidx = (
        lax.bitwise_and(t32, -VBLK)
        + lax.bitwise_and(t32, PIECE - 1) * 8
        + lax.shift_right_logical(lax.bitwise_and(t32, VBLK - 1), 11)
    )
    hidx2d = hidx.reshape(NW * NCHUNK, ROWS_PER_CHUNK)

    w16 = jnp.zeros((D, 16), jnp.float32)
    w16 = w16.at[:, :NUM_CLASS].set(fc_w.T / L)
    w16 = w16.at[:, 8:8 + NUM_CLASS].set(fc_w.T / L)
    w_pad = jnp.tile(w16, (1, 8))
    b16 = jnp.zeros((1, 16), jnp.float32)
    b16 = b16.at[0, :NUM_CLASS].set(fc_b / L)
    b16 = b16.at[0, 8:8 + NUM_CLASS].set(fc_b / L)
    b_pad = jnp.tile(b16, (1, 8))

    proj = _tc_proj(emb_table.T, w_pad, b_pad)
    proj16 = proj.reshape(PACK_ROWS * 8, 16)

    packed = _sc_bag_sums(hidx2d, proj16)
    return packed.reshape(B, 8)[:, :NUM_CLASS]


# speedup vs baseline: 3.7948x; 1.2015x over previous
import functools

import jax
import jax.numpy as jnp
from jax import lax
from jax.experimental import pallas as pl
from jax.experimental.pallas import tpu as pltpu
from jax.experimental.pallas import tpu_sc as plsc

VOCAB = 1000000
D = 64
B = 16384
L = 50
NUM_CLASS = 5

NC = 2
NS = 16
NW = NC * NS
BAGS_PER_W = B // NW
BAGS_PER_CHUNK = 2
ROWS_PER_CHUNK = BAGS_PER_CHUNK * L
NCHUNK = BAGS_PER_W // BAGS_PER_CHUNK
NBUF = 8

VBLK = 16384
N_VBLK = -(-VOCAB // VBLK)
PIECE = VBLK // 8
PACK_ROWS = N_VBLK * (VBLK // 8)


def _tc_proj_body(x_ref, w_ref, b_ref, o_ref):
    y = lax.dot_general(x_ref[...].astype(jnp.bfloat16),
                        w_ref[...].astype(jnp.bfloat16),
                        (((0,), (0,)), ((), ())),
                        preferred_element_type=jnp.float32)
    y = y + b_ref[...]
    lane = lax.broadcasted_iota(jnp.int32, (PIECE, 128), 1)
    slot = lax.shift_right_logical(lane, 4)
    acc = jnp.zeros((PIECE, 128), jnp.float32)
    for k in range(8):
        acc = acc + jnp.where(slot == k, y[PIECE * k:PIECE * (k + 1), :], 0.0)
    o_ref[...] = acc


def _tc_proj(table_t, w_pad, b_pad):
    return pl.pallas_call(
        _tc_proj_body,
        grid=(N_VBLK,),
        in_specs=[
            pl.BlockSpec((D, VBLK), lambda i: (0, i)),
            pl.BlockSpec((D, 128), lambda i: (0, 0)),
            pl.BlockSpec((1, 128), lambda i: (0, 0)),
        ],
        out_specs=pl.BlockSpec((VBLK // 8, 128), lambda i: (i, 0)),
        out_shape=jax.ShapeDtypeStruct((PACK_ROWS, 128), jnp.float32),
        compiler_params=pltpu.CompilerParams(
            fuse_transposed_lhs_in_matmul=True),
    )(table_t, w_pad, b_pad)


def _sc_body(idx_hbm, proj_hbm, out_hbm, idx_v, gbuf, outb, sem0, sem1,
             sem2, sem3, sem4, sem5, sem6, sem7):
    wid = lax.axis_index("s") * NC + lax.axis_index("c")
    row0 = wid * NCHUNK
    pltpu.sync_copy(idx_hbm.at[pl.ds(row0, NCHUNK)], idx_v)

    sems = (sem0, sem1, sem2, sem3, sem4, sem5, sem6, sem7)

    def start(j, b):
        pltpu.async_copy(proj_hbm.at[idx_v.at[j]], gbuf.at[b], sems[b])

    def wait(b):
        pltpu.make_async_copy(proj_hbm.at[idx_v.at[0]], gbuf.at[b], sems[b]).wait()

    for b in range(NBUF):
        start(b, b)

    zeros = jnp.zeros((16,), jnp.float32)
    iota16 = lax.iota(jnp.int32, 16)
    lo_mask = iota16 < 8
    perm_idx = lax.rem(iota16, 8)

    @pl.loop(0, NCHUNK, step=NBUF)
    def _chunks(g):
        for b in range(NBUF):
            j = g + b
            wait(b)

            accs = []
            for bic in range(BAGS_PER_CHUNK):
                r0 = bic * L

                def body(r, acc):
                    return acc + gbuf[b, r0 + r, pl.ds(0, 16)]

                accs.append(lax.fori_loop(0, L, body, zeros, unroll=4))

            hi = lax.gather(
                accs[1], perm_idx[:, None],
                lax.GatherDimensionNumbers(offset_dims=(),
                                           collapsed_slice_dims=(0,),
                                           start_index_map=(0,)),
                slice_sizes=(1,),
                mode=lax.GatherScatterMode.PROMISE_IN_BOUNDS)
            combined = jnp.where(lo_mask, accs[0], hi)
            outb[lax.shift_right_logical(j, 3),
                 pl.ds(lax.bitwise_and(j, 7) * 16, 16)] = combined

            @pl.when(j + NBUF < NCHUNK)
            def _():
                start(j + NBUF, b)

    pltpu.sync_copy(outb, out_hbm.at[pl.ds(wid * (BAGS_PER_W // 16),
                                           BAGS_PER_W // 16)])


@functools.partial(
    pl.kernel,
    out_type=jax.ShapeDtypeStruct((B // 16, 128), jnp.float32),
    mesh=plsc.VectorSubcoreMesh(core_axis_name="c", subcore_axis_name="s",
                                num_cores=NC, num_subcores=NS),
    scratch_types=[
        pltpu.VMEM((NCHUNK, ROWS_PER_CHUNK), jnp.int32),
        pltpu.VMEM((NBUF, ROWS_PER_CHUNK, 16), jnp.float32),
        pltpu.VMEM((BAGS_PER_W // 16, 128), jnp.float32),
        pltpu.SemaphoreType.DMA,
        pltpu.SemaphoreType.DMA,
        pltpu.SemaphoreType.DMA,
        pltpu.SemaphoreType.DMA,
        pltpu.SemaphoreType.DMA,
        pltpu.SemaphoreType.DMA,
        pltpu.SemaphoreType.DMA,
        pltpu.SemaphoreType.DMA,
    ],
    compiler_params=pltpu.CompilerParams(use_tc_tiling_on_sc=False),
)
def _sc_bag_sums(idx_hbm, proj_hbm, out_hbm, idx_v, gbuf, outb, sem0, sem1,
                 sem2, sem3, sem4, sem5, sem6, sem7):
    _sc_body(idx_hbm, proj_hbm, out_hbm, idx_v, gbuf, outb, sem0, sem1,
             sem2, sem3, sem4, sem5, sem6, sem7)


def kernel(text, emb_table, fc_w, fc_b):
    t32 = text.astype(jnp.int32)
    hidx = (
        lax.bitwise_and(t32, -VBLK)
        + lax.bitwise_and(t32, PIECE - 1) * 8
        + lax.shift_right_logical(lax.bitwise_and(t32, VBLK - 1), 11)
    )
    hidx2d = hidx.reshape(NW * NCHUNK, ROWS_PER_CHUNK)

    w16 = jnp.zeros((D, 16), jnp.float32)
    w16 = w16.at[:, :NUM_CLASS].set(fc_w.T / L)
    w16 = w16.at[:, 8:8 + NUM_CLASS].set(fc_w.T / L)
    w_pad = jnp.tile(w16, (1, 8))
    b16 = jnp.zeros((1, 16), jnp.float32)
    b16 = b16.at[0, :NUM_CLASS].set(fc_b / L)
    b16 = b16.at[0, 8:8 + NUM_CLASS].set(fc_b / L)
    b_pad = jnp.tile(b16, (1, 8))

    proj = _tc_proj(emb_table.T, w_pad, b_pad)
    proj16 = proj.reshape(PACK_ROWS * 8, 16)

    packed = _sc_bag_sums(hidx2d, proj16)
    return packed.reshape(B, 8)[:, :NUM_CLASS]
